# baseline (device time: 85683 ns/iter reference)
import jax
import jax.numpy as jnp
from jax import lax
from jax.experimental import pallas as pl
from jax.experimental.pallas import tpu as pltpu

N_DEV = 8
PERMS = ((1, 3, 4), (3, 4, 1), (4, 1, 3))
COLS = ((0, 640), (640, 1344), (1344, 2048))


def kernel(x, w_mat, scale_x, scale_w):
    m, k_per = x.shape
    _, n = w_mat.shape
    m_per = m // N_DEV

    def body(x_ref, w_ref, sx_ref, sw_ref, out_ref,
             acc0, acc1, acc2, r1_0, r1_1, r1_2, r2_0, r2_1, r2_2,
             r3_0, r3_1, r3_2, send_sems, recv_sems):
        accs = (acc0, acc1, acc2)
        recv1 = (r1_0, r1_1, r1_2)
        recv2 = (r2_0, r2_1, r2_2)
        recv3 = (r3_0, r3_1, r3_2)
        my = lax.axis_index("i")

        barrier_sem = pltpu.get_barrier_semaphore()
        for mask in (1, 3, 4):
            pl.semaphore_signal(barrier_sem, inc=1, device_id=(my ^ mask,),
                                device_id_type=pl.DeviceIdType.MESH)
        pl.semaphore_wait(barrier_sem, 3)

        w = w_ref[...].astype(jnp.bfloat16)

        def partial(c, g):
            xs = x_ref[pl.ds(c * m_per, m_per), :].astype(jnp.bfloat16)
            wg = w[:, COLS[g][0]:COLS[g][1]]
            return lax.dot_general(xs, wg, (((1,), (0,)), ((), ())),
                                   preferred_element_type=jnp.float32)

        def chunk_of(g, code):
            b1, b2, b3 = (code >> 2) & 1, (code >> 1) & 1, code & 1
            m1, m2, m3 = PERMS[g]
            return my ^ (b1 * m1 ^ b2 * m2 ^ b3 * m3)

        def exchange(g, step, src, dst):
            rdma = pltpu.make_async_remote_copy(
                src_ref=src, dst_ref=dst,
                send_sem=send_sems.at[3 * g + step],
                recv_sem=recv_sems.at[3 * g + step],
                device_id=(my ^ PERMS[g][step],),
                device_id_type=pl.DeviceIdType.MESH,
            )
            rdma.start()
            return rdma

        def add(a, b):
            return (a.astype(jnp.float32) + b.astype(jnp.float32)
                    ).astype(jnp.bfloat16)

        rd1 = []
        for g in range(3):
            for code in range(4, 8):
                accs[g][code] = partial(chunk_of(g, code), g).astype(jnp.bfloat16)
            rd1.append(exchange(g, 0, accs[g].at[4:8], recv1[g]))
        for g in range(3):
            for code in range(0, 4):
                accs[g][code] = partial(chunk_of(g, code), g).astype(jnp.bfloat16)

        rd2 = []
        for g in range(3):
            rd1[g].wait()
            accs[g][2:4] = add(accs[g][2:4], recv1[g][2:4])
            rd2.append(exchange(g, 1, accs[g].at[2:4], recv2[g]))
        for g in range(3):
            accs[g][0:2] = add(accs[g][0:2], recv1[g][0:2])

        rd3 = []
        for g in range(3):
            rd2[g].wait()
            accs[g][1] = add(accs[g][1], recv2[g][1])
            rd3.append(exchange(g, 2, accs[g].at[1], recv3[g]))
        for g in range(3):
            accs[g][0] = add(accs[g][0], recv2[g][0])

        scale = sx_ref[0] * sw_ref[0]
        for g in range(3):
            rd3[g].wait()
            out_ref[:, COLS[g][0]:COLS[g][1]] = (
                accs[g][0].astype(jnp.float32)
                + recv3[g][...].astype(jnp.float32)
            ) * scale

    widths = [hi - lo for lo, hi in COLS]
    return pl.pallas_call(
        body,
        out_shape=jax.ShapeDtypeStruct((m_per, n), jnp.float32),
        in_specs=[
            pl.BlockSpec(memory_space=pltpu.VMEM),
            pl.BlockSpec(memory_space=pltpu.VMEM),
            pl.BlockSpec(memory_space=pltpu.SMEM),
            pl.BlockSpec(memory_space=pltpu.SMEM),
        ],
        out_specs=pl.BlockSpec(memory_space=pltpu.VMEM),
        scratch_shapes=[
            *[pltpu.VMEM((N_DEV, m_per, wd), jnp.bfloat16) for wd in widths],
            *[pltpu.VMEM((4, m_per, wd), jnp.bfloat16) for wd in widths],
            *[pltpu.VMEM((2, m_per, wd), jnp.bfloat16) for wd in widths],
            *[pltpu.VMEM((m_per, wd), jnp.bfloat16) for wd in widths],
            pltpu.SemaphoreType.DMA((9,)),
            pltpu.SemaphoreType.DMA((9,)),
        ],
        compiler_params=pltpu.CompilerParams(
            collective_id=0, vmem_limit_bytes=100 * 1024 * 1024,
        ),
    )(x, w_mat, scale_x, scale_w)


# device time: 72842 ns/iter; 1.1763x vs baseline; 1.1763x over previous
import jax
import jax.numpy as jnp
from jax import lax
from jax.experimental import pallas as pl
from jax.experimental.pallas import tpu as pltpu

N_DEV = 8
PERMS = ((1, 3, 4), (3, 4, 1), (4, 1, 3))
COLS = ((0, 768), (768, 1408), (1408, 2048))
N_MSG = 5


def kernel(x, w_mat, scale_x, scale_w):
    m, k_per = x.shape
    _, n = w_mat.shape
    m_per = m // N_DEV

    def body(x_ref, w_ref, sx_ref, sw_ref, out_ref,
             acc0, acc1, acc2, r1_0, r1_1, r1_2, r2_0, r2_1, r2_2,
             r3_0, r3_1, r3_2, send_sems, recv_sems):
        accs = (acc0, acc1, acc2)
        recv1 = (r1_0, r1_1, r1_2)
        recv2 = (r2_0, r2_1, r2_2)
        recv3 = (r3_0, r3_1, r3_2)
        my = lax.axis_index("i")

        barrier_sem = pltpu.get_barrier_semaphore()
        for mask in (1, 3, 4):
            pl.semaphore_signal(barrier_sem, inc=1, device_id=(my ^ mask,),
                                device_id_type=pl.DeviceIdType.MESH)
        pl.semaphore_wait(barrier_sem, 3)

        w = w_ref[...].astype(jnp.bfloat16)

        def partial(g, code):
            b1, b2, b3 = (code >> 2) & 1, (code >> 1) & 1, code & 1
            m1, m2, m3 = PERMS[g]
            c = my ^ (b1 * m1 ^ b2 * m2 ^ b3 * m3)
            xs = x_ref[pl.ds(c * m_per, m_per), :].astype(jnp.bfloat16)
            wg = w[:, COLS[g][0]:COLS[g][1]]
            accs[g][code] = lax.dot_general(
                xs, wg, (((1,), (0,)), ((), ())),
                preferred_element_type=jnp.float32,
            ).astype(jnp.bfloat16)

        def exchange(g, step, msg, src, dst):
            rdma = pltpu.make_async_remote_copy(
                src_ref=src, dst_ref=dst,
                send_sem=send_sems.at[N_MSG * g + msg],
                recv_sem=recv_sems.at[N_MSG * g + msg],
                device_id=(my ^ PERMS[g][step],),
                device_id_type=pl.DeviceIdType.MESH,
            )
            rdma.start()
            return rdma

        def add(a, b):
            return (a.astype(jnp.float32) + b.astype(jnp.float32)
                    ).astype(jnp.bfloat16)

        ex1a, ex1b, ex2a, ex2b, ex3 = [], [], [], [], []
        for g in range(3):
            partial(g, 6)
            partial(g, 7)
            ex1a.append(exchange(g, 0, 0, accs[g].at[6:8], recv1[g].at[2:4]))
        for g in range(3):
            partial(g, 4)
            partial(g, 5)
            ex1b.append(exchange(g, 0, 1, accs[g].at[4:6], recv1[g].at[0:2]))
        for g in range(3):
            partial(g, 2)
            partial(g, 3)
        for g in range(3):
            partial(g, 0)
            partial(g, 1)

        for g in range(3):
            ex1a[g].wait_recv()
            accs[g][3] = add(accs[g][3], recv1[g][3])
            ex2a.append(exchange(g, 1, 2, accs[g].at[3], recv2[g].at[1]))
            accs[g][2] = add(accs[g][2], recv1[g][2])
            ex2b.append(exchange(g, 1, 3, accs[g].at[2], recv2[g].at[0]))
        for g in range(3):
            ex1b[g].wait_recv()
            accs[g][1] = add(accs[g][1], recv1[g][1])

        for g in range(3):
            ex2a[g].wait_recv()
            accs[g][1] = add(accs[g][1], recv2[g][1])
            ex3.append(exchange(g, 2, 4, accs[g].at[1], recv3[g]))
        for g in range(3):
            accs[g][0] = add(accs[g][0], recv1[g][0])
            ex2b[g].wait_recv()
            accs[g][0] = add(accs[g][0], recv2[g][0])

        scale = sx_ref[0] * sw_ref[0]
        for g in range(3):
            ex3[g].wait_recv()
            out_ref[:, COLS[g][0]:COLS[g][1]] = (
                accs[g][0].astype(jnp.float32)
                + recv3[g][...].astype(jnp.float32)
            ) * scale

        for rd in (*ex1a, *ex1b, *ex2a, *ex2b, *ex3):
            rd.wait_send()

    widths = [hi - lo for lo, hi in COLS]
    return pl.pallas_call(
        body,
        out_shape=jax.ShapeDtypeStruct((m_per, n), jnp.float32),
        in_specs=[
            pl.BlockSpec(memory_space=pltpu.VMEM),
            pl.BlockSpec(memory_space=pltpu.VMEM),
            pl.BlockSpec(memory_space=pltpu.SMEM),
            pl.BlockSpec(memory_space=pltpu.SMEM),
        ],
        out_specs=pl.BlockSpec(memory_space=pltpu.VMEM),
        scratch_shapes=[
            *[pltpu.VMEM((N_DEV, m_per, wd), jnp.bfloat16) for wd in widths],
            *[pltpu.VMEM((4, m_per, wd), jnp.bfloat16) for wd in widths],
            *[pltpu.VMEM((2, m_per, wd), jnp.bfloat16) for wd in widths],
            *[pltpu.VMEM((m_per, wd), jnp.bfloat16) for wd in widths],
            pltpu.SemaphoreType.DMA((3 * N_MSG,)),
            pltpu.SemaphoreType.DMA((3 * N_MSG,)),
        ],
        compiler_params=pltpu.CompilerParams(
            collective_id=0, vmem_limit_bytes=100 * 1024 * 1024,
        ),
    )(x, w_mat, scale_x, scale_w)
